# baseline (speedup 1.0000x reference)
import functools

import jax
import jax.numpy as jnp
from jax import lax
from jax.experimental import pallas as pl
from jax.experimental.pallas import tpu as pltpu
from jax.experimental.pallas import tpu_sc as plsc

B = 4096
S = 200
DIM = 128
N = B * S
NPP = 3 * S

_info = plsc.get_sparse_core_info()
NC, NS, L = _info.num_cores, _info.num_subcores, _info.num_lanes
NW = NC * NS
ROWS_PER_W = N // NW
GR = 128
NG = ROWS_PER_W // GR
IDROWS = ROWS_PER_W // GR


def _build_pospad_tc(padding_embedding, pos_embedding):
    def body(pad_ref, pos_ref, out_ref):
        out_ref[...] = pad_ref[...][:, None, :] + pos_ref[0:S][None, :, :]

    return pl.pallas_call(
        body,
        out_shape=jax.ShapeDtypeStruct((3, S, DIM), jnp.float32),
    )(padding_embedding, pos_embedding)


def _sc_lookup(embedding, pospad, char2d, pad2d):
    mesh = plsc.VectorSubcoreMesh(core_axis_name="c", subcore_axis_name="s")

    @functools.partial(
        pl.kernel,
        mesh=mesh,
        out_type=jax.ShapeDtypeStruct((N, DIM), jnp.float32),
        scratch_types=[
            pltpu.VMEM((IDROWS, GR), jnp.int32),
            pltpu.VMEM((IDROWS, GR), jnp.int32),
            pltpu.VMEM((2, GR, DIM), jnp.float32),
            pltpu.VMEM((2, GR, DIM), jnp.float32),
            pltpu.SemaphoreType.DMA,
            pltpu.SemaphoreType.DMA,
        ],
    )
    def k(emb_hbm, pp_hbm, char_hbm, pad_hbm, out_hbm,
          char_v, ppidx_v, bufa, bufb, sem_g, sem_o):
        wid = lax.axis_index("s") * NC + lax.axis_index("c")
        w_row0 = wid * ROWS_PER_W
        idrow0 = pl.multiple_of(w_row0 // GR, IDROWS)

        pltpu.sync_copy(char_hbm.at[pl.ds(idrow0, IDROWS)], char_v)
        pltpu.sync_copy(pad_hbm.at[pl.ds(idrow0, IDROWS)], ppidx_v)

        lane = lax.iota(jnp.int32, L)

        def idx_body(j, _):
            r = j // (GR // L)
            c = j % (GR // L)
            cs = pl.ds(c * L, L)
            base = (w_row0 + j * L).astype(jnp.int32)
            ppidx_v[r, cs] = ppidx_v[r, cs] * S + (base + lane) % S
            return _
        lax.fori_loop(0, ROWS_PER_W // L, idx_body, 0)

        pltpu.async_copy(emb_hbm.at[char_v.at[0]], bufa.at[0], sem_g)
        pltpu.async_copy(pp_hbm.at[ppidx_v.at[0]], bufb.at[0], sem_g)

        def group_body(g, _):
            sp = g % 2

            @pl.when(jnp.logical_and(g >= 1, g < NG - 1))
            def _wait_out():
                pltpu.make_async_copy(
                    bufa.at[1 - sp],
                    out_hbm.at[pl.ds(w_row0, GR)], sem_o).wait()

            pltpu.make_async_copy(emb_hbm.at[char_v.at[0]],
                                  bufa.at[sp], sem_g).wait()
            pltpu.make_async_copy(pp_hbm.at[ppidx_v.at[0]],
                                  bufb.at[sp], sem_g).wait()

            @pl.when(g < NG - 1)
            def _next_gather():
                pltpu.async_copy(emb_hbm.at[char_v.at[g + 1]],
                                 bufa.at[1 - sp], sem_g)
                pltpu.async_copy(pp_hbm.at[ppidx_v.at[g + 1]],
                                 bufb.at[1 - sp], sem_g)

            def add_row(r, _):
                for c in range(DIM // L):
                    cs = pl.ds(c * L, L)
                    bufa[sp, r, cs] = bufa[sp, r, cs] + bufb[sp, r, cs]
                return _
            lax.fori_loop(0, GR, add_row, 0)

            pltpu.async_copy(
                bufa.at[sp], out_hbm.at[pl.ds(w_row0 + g * GR, GR)],
                sem_o)
            return _
        lax.fori_loop(0, NG, group_body, 0)

        for _d in range(2):
            pltpu.make_async_copy(
                bufa.at[0], out_hbm.at[pl.ds(w_row0, GR)], sem_o).wait()

    return k(embedding, pospad, char2d, pad2d)


def kernel(char_ids, pad_ids, embedding, pos_embedding, padding_embedding):
    pospad = _build_pospad_tc(padding_embedding, pos_embedding)
    pospad = pospad.reshape(NPP, DIM)
    char2d = char_ids.reshape(N // GR, GR).astype(jnp.int32)
    pad2d = pad_ids.reshape(N // GR, GR).astype(jnp.int32)
    out = _sc_lookup(embedding, pospad, char2d, pad2d)
    return out.reshape(B, S, DIM)


# speedup vs baseline: 1.0001x; 1.0001x over previous
import functools

import jax
import jax.numpy as jnp
from jax import lax
from jax.experimental import pallas as pl
from jax.experimental.pallas import tpu as pltpu
from jax.experimental.pallas import tpu_sc as plsc

B = 4096
S = 200
DIM = 128
N = B * S
NPP = 3 * S

_info = plsc.get_sparse_core_info()
NC, NS, L = _info.num_cores, _info.num_subcores, _info.num_lanes
NW = NC * NS
ROWS_PER_W = N // NW
GR = 128
NG = ROWS_PER_W // GR
IDROWS = ROWS_PER_W // GR


def _build_pospad_tc(padding_embedding, pos_embedding):
    def body(pad_ref, pos_ref, out_ref):
        out_ref[...] = pad_ref[...][:, None, :] + pos_ref[0:S][None, :, :]

    return pl.pallas_call(
        body,
        out_shape=jax.ShapeDtypeStruct((3, S, DIM), jnp.float32),
    )(padding_embedding, pos_embedding)


def _sc_lookup(embedding, pospad, char2d, pad2d):
    mesh = plsc.VectorSubcoreMesh(core_axis_name="c", subcore_axis_name="s")

    @functools.partial(
        pl.kernel,
        mesh=mesh,
        out_type=jax.ShapeDtypeStruct((N, DIM), jnp.float32),
        scratch_types=[
            pltpu.VMEM((IDROWS, GR), jnp.int32),
            pltpu.VMEM((IDROWS, GR), jnp.int32),
            pltpu.VMEM((2, GR, DIM), jnp.float32),
            pltpu.VMEM((2, GR, DIM), jnp.float32),
            pltpu.SemaphoreType.DMA,
            pltpu.SemaphoreType.DMA,
        ],
    )
    def k(emb_hbm, pp_hbm, char_hbm, pad_hbm, out_hbm,
          char_v, ppidx_v, bufa, bufb, sem_g, sem_o):
        wid = lax.axis_index("s") * NC + lax.axis_index("c")
        w_row0 = wid * ROWS_PER_W
        idrow0 = pl.multiple_of(w_row0 // GR, IDROWS)

        pltpu.sync_copy(char_hbm.at[pl.ds(idrow0, IDROWS)], char_v)
        pltpu.sync_copy(pad_hbm.at[pl.ds(idrow0, IDROWS)], ppidx_v)

        lane = lax.iota(jnp.int32, L)

        def idx_body(j, _):
            r = j // (GR // L)
            c = j % (GR // L)
            cs = pl.ds(c * L, L)
            base = (w_row0 + j * L).astype(jnp.int32)
            ppidx_v[r, cs] = ppidx_v[r, cs] * S + (base + lane) % S
            return _
        lax.fori_loop(0, ROWS_PER_W // L, idx_body, 0)

        pltpu.async_copy(emb_hbm.at[char_v.at[0]], bufa.at[0], sem_g)
        pltpu.async_copy(pp_hbm.at[ppidx_v.at[0]], bufb.at[0], sem_g)

        def group_body(g, _):
            sp = g % 2

            @pl.when(jnp.logical_and(g >= 1, g < NG - 1))
            def _wait_out():
                pltpu.make_async_copy(
                    bufa.at[1 - sp],
                    out_hbm.at[pl.ds(w_row0, GR)], sem_o).wait()

            @pl.when(g < NG - 1)
            def _next_gather():
                pltpu.async_copy(emb_hbm.at[char_v.at[g + 1]],
                                 bufa.at[1 - sp], sem_g)
                pltpu.async_copy(pp_hbm.at[ppidx_v.at[g + 1]],
                                 bufb.at[1 - sp], sem_g)

            pltpu.make_async_copy(emb_hbm.at[char_v.at[0]],
                                  bufa.at[sp], sem_g).wait()
            pltpu.make_async_copy(pp_hbm.at[ppidx_v.at[0]],
                                  bufb.at[sp], sem_g).wait()

            def add_row(r, _):
                for c in range(DIM // L):
                    cs = pl.ds(c * L, L)
                    bufa[sp, r, cs] = bufa[sp, r, cs] + bufb[sp, r, cs]
                return _
            lax.fori_loop(0, GR, add_row, 0)

            pltpu.async_copy(
                bufa.at[sp], out_hbm.at[pl.ds(w_row0 + g * GR, GR)],
                sem_o)
            return _
        lax.fori_loop(0, NG, group_body, 0)

        for _d in range(2):
            pltpu.make_async_copy(
                bufa.at[0], out_hbm.at[pl.ds(w_row0, GR)], sem_o).wait()

    return k(embedding, pospad, char2d, pad2d)


def kernel(char_ids, pad_ids, embedding, pos_embedding, padding_embedding):
    pospad = _build_pospad_tc(padding_embedding, pos_embedding)
    pospad = pospad.reshape(NPP, DIM)
    char2d = char_ids.reshape(N // GR, GR).astype(jnp.int32)
    pad2d = pad_ids.reshape(N // GR, GR).astype(jnp.int32)
    out = _sc_lookup(embedding, pospad, char2d, pad2d)
    return out.reshape(B, S, DIM)


# speedup vs baseline: 1.0979x; 1.0978x over previous
import functools

import jax
import jax.numpy as jnp
from jax import lax
from jax.experimental import pallas as pl
from jax.experimental.pallas import tpu as pltpu
from jax.experimental.pallas import tpu_sc as plsc

B = 4096
S = 200
DIM = 128
N = B * S
NPP = 3 * S

_info = plsc.get_sparse_core_info()
NC, NS, L = _info.num_cores, _info.num_subcores, _info.num_lanes
NW = NC * NS
ROWS_PER_W = N // NW
GR = 128
NG = ROWS_PER_W // GR
IDROWS = ROWS_PER_W // GR


def _build_pospad_tc(padding_embedding, pos_embedding):
    def body(pad_ref, pos_ref, out_ref):
        out_ref[...] = pad_ref[...][:, None, :] + pos_ref[0:S][None, :, :]

    return pl.pallas_call(
        body,
        out_shape=jax.ShapeDtypeStruct((3, S, DIM), jnp.float32),
    )(padding_embedding, pos_embedding)


def _pack_ids_tc(char2d, pad2d):
    def body(char_ref, pad_ref, out_ref):
        out_ref[...] = char_ref[...] * 4 + pad_ref[...]

    return pl.pallas_call(
        body,
        out_shape=jax.ShapeDtypeStruct(char2d.shape, jnp.int32),
    )(char2d, pad2d)


def _sc_lookup(embedding, pospad, comb2d):
    mesh = plsc.VectorSubcoreMesh(core_axis_name="c", subcore_axis_name="s")

    @functools.partial(
        pl.kernel,
        mesh=mesh,
        out_type=jax.ShapeDtypeStruct((N, DIM), jnp.float32),
        scratch_types=[
            pltpu.VMEM((IDROWS, GR), jnp.int32),
            pltpu.VMEM((2, GR), jnp.int32),
            pltpu.VMEM((2, GR), jnp.int32),
            pltpu.VMEM((3, GR, DIM), jnp.float32),
            pltpu.VMEM((2, GR, DIM), jnp.float32),
            pltpu.SemaphoreType.DMA,
            pltpu.SemaphoreType.DMA,
        ],
    )
    def k(emb_hbm, pp_hbm, comb_hbm, out_hbm,
          comb_v, cidx_v, pidx_v, bufa, bufb, sem_g, sem_o):
        wid = lax.axis_index("s") * NC + lax.axis_index("c")
        w_row0 = wid * ROWS_PER_W
        idrow0 = pl.multiple_of(w_row0 // GR, IDROWS)

        pltpu.sync_copy(comb_hbm.at[pl.ds(idrow0, IDROWS)], comb_v)

        lane = lax.iota(jnp.int32, L)

        def build_idx(g, islot):
            for c in range(GR // L):
                cs = pl.ds(c * L, L)
                w = comb_v[g, cs]
                base = (w_row0 + c * L).astype(jnp.int32) + g * GR
                cidx_v[islot, cs] = w >> 2
                pidx_v[islot, cs] = (w & 3) * S + (base + lane) % S

        def gather(g, islot, aslot, bslot):
            pltpu.async_copy(emb_hbm.at[cidx_v.at[islot]],
                             bufa.at[aslot], sem_g)
            pltpu.async_copy(pp_hbm.at[pidx_v.at[islot]],
                             bufb.at[bslot], sem_g)

        build_idx(0, 0)
        gather(0, 0, 0, 0)

        def group_body(g, _):
            sa = g % 3
            sb = g % 2

            @pl.when(jnp.logical_and(g >= 2, g < NG - 1))
            def _wait_out():
                pltpu.make_async_copy(
                    bufa.at[0], out_hbm.at[pl.ds(w_row0, GR)], sem_o).wait()

            @pl.when(g < NG - 1)
            def _next_gather():
                build_idx(g + 1, (g + 1) % 2)
                gather(g + 1, (g + 1) % 2, (g + 1) % 3, (g + 1) % 2)

            pltpu.make_async_copy(emb_hbm.at[cidx_v.at[0]],
                                  bufa.at[sa], sem_g).wait()
            pltpu.make_async_copy(pp_hbm.at[pidx_v.at[0]],
                                  bufb.at[sb], sem_g).wait()

            def add_row(r, _):
                for c in range(DIM // L):
                    cs = pl.ds(c * L, L)
                    bufa[sa, r, cs] = bufa[sa, r, cs] + bufb[sb, r, cs]
                return _
            lax.fori_loop(0, GR, add_row, 0)

            pltpu.async_copy(
                bufa.at[sa], out_hbm.at[pl.ds(w_row0 + g * GR, GR)],
                sem_o)
            return _
        lax.fori_loop(0, NG, group_body, 0)

        for _d in range(3):
            pltpu.make_async_copy(
                bufa.at[0], out_hbm.at[pl.ds(w_row0, GR)], sem_o).wait()

    return k(embedding, pospad, comb2d)


def kernel(char_ids, pad_ids, embedding, pos_embedding, padding_embedding):
    pospad = _build_pospad_tc(padding_embedding, pos_embedding)
    pospad = pospad.reshape(NPP, DIM)
    char2d = char_ids.reshape(N // GR, GR).astype(jnp.int32)
    pad2d = pad_ids.reshape(N // GR, GR).astype(jnp.int32)
    comb2d = _pack_ids_tc(char2d, pad2d)
    out = _sc_lookup(embedding, pospad, comb2d)
    return out.reshape(B, S, DIM)


# speedup vs baseline: 1.1083x; 1.0095x over previous
import functools

import jax
import jax.numpy as jnp
from jax import lax
from jax.experimental import pallas as pl
from jax.experimental.pallas import tpu as pltpu
from jax.experimental.pallas import tpu_sc as plsc

B = 4096
S = 200
DIM = 128
N = B * S
NPP = 3 * S

_info = plsc.get_sparse_core_info()
NC, NS, L = _info.num_cores, _info.num_subcores, _info.num_lanes
NW = NC * NS
ROWS_PER_W = N // NW
GR = 128
NG = ROWS_PER_W // GR
IDROWS = ROWS_PER_W // GR


def _build_pospad_tc(padding_embedding, pos_embedding):
    def body(pad_ref, pos_ref, out_ref):
        out_ref[...] = pad_ref[...][:, None, :] + pos_ref[0:S][None, :, :]

    return pl.pallas_call(
        body,
        out_shape=jax.ShapeDtypeStruct((3, S, DIM), jnp.float32),
    )(padding_embedding, pos_embedding)


def _pack_ids_tc(char2d, pad2d):
    def body(char_ref, pad_ref, out_ref):
        out_ref[...] = char_ref[...] * 4 + pad_ref[...]

    return pl.pallas_call(
        body,
        out_shape=jax.ShapeDtypeStruct(char2d.shape, jnp.int32),
    )(char2d, pad2d)


def _sc_lookup(embedding, pospad, comb2d):
    mesh = plsc.VectorSubcoreMesh(core_axis_name="c", subcore_axis_name="s")

    @functools.partial(
        pl.kernel,
        mesh=mesh,
        out_type=jax.ShapeDtypeStruct((N, DIM), jnp.float32),
        scratch_types=[
            pltpu.VMEM((IDROWS, GR), jnp.int32),
            pltpu.VMEM((2, GR), jnp.int32),
            pltpu.VMEM((2, GR), jnp.int32),
            pltpu.VMEM((3, GR, DIM), jnp.float32),
            pltpu.VMEM((2, GR, DIM), jnp.float32),
            pltpu.SemaphoreType.DMA,
            pltpu.SemaphoreType.DMA,
        ],
    )
    def k(emb_hbm, pp_hbm, comb_hbm, out_hbm,
          comb_v, cidx_v, pidx_v, bufa, bufb, sem_g, sem_o):
        wid = lax.axis_index("s") * NC + lax.axis_index("c")
        w_row0 = wid * ROWS_PER_W
        idrow0 = pl.multiple_of(w_row0 // GR, IDROWS)

        pltpu.sync_copy(comb_hbm.at[pl.ds(idrow0, IDROWS)], comb_v)

        lane = lax.iota(jnp.int32, L)

        def build_idx(g, islot):
            for c in range(GR // L):
                cs = pl.ds(c * L, L)
                w = comb_v[g, cs]
                base = (w_row0 + c * L).astype(jnp.int32) + g * GR
                cidx_v[islot, cs] = w >> 2
                pidx_v[islot, cs] = (w & 3) * S + (base + lane) % S

        def gather(g, islot, aslot, bslot):
            pltpu.async_copy(emb_hbm.at[cidx_v.at[islot]],
                             bufa.at[aslot], sem_g)
            pltpu.async_copy(pp_hbm.at[pidx_v.at[islot]],
                             bufb.at[bslot], sem_g)

        def wait_gather(aslot, bslot):
            pltpu.make_async_copy(emb_hbm.at[cidx_v.at[0]],
                                  bufa.at[aslot], sem_g).wait()
            pltpu.make_async_copy(pp_hbm.at[pidx_v.at[0]],
                                  bufb.at[bslot], sem_g).wait()

        def add_group(aslot, bslot):
            def add_row(r, _):
                for c in range(DIM // L):
                    cs = pl.ds(c * L, L)
                    bufa[aslot, r, cs] = (bufa[aslot, r, cs]
                                          + bufb[bslot, r, cs])
                return _
            lax.fori_loop(0, GR, add_row, 0)

        def copy_out(g, aslot):
            pltpu.async_copy(
                bufa.at[aslot], out_hbm.at[pl.ds(w_row0 + g * GR, GR)],
                sem_o)

        def wait_out():
            pltpu.make_async_copy(
                bufa.at[0], out_hbm.at[pl.ds(w_row0, GR)], sem_o).wait()

        build_idx(0, 0)
        gather(0, 0, 0, 0)
        for g in (1, 2):
            build_idx(g, g % 2)
            gather(g, g % 2, g % 3, g % 2)
            wait_gather((g - 1) % 3, (g - 1) % 2)
            add_group((g - 1) % 3, (g - 1) % 2)
            copy_out(g - 1, (g - 1) % 3)

        def group_body(g, _):
            wait_out()
            build_idx(g, g % 2)
            gather(g, g % 2, g % 3, g % 2)
            wait_gather((g - 1) % 3, (g - 1) % 2)
            add_group((g - 1) % 3, (g - 1) % 2)
            copy_out(g - 1, (g - 1) % 3)
            return _
        lax.fori_loop(3, NG, group_body, 0)

        wait_gather((NG - 1) % 3, (NG - 1) % 2)
        add_group((NG - 1) % 3, (NG - 1) % 2)
        copy_out(NG - 1, (NG - 1) % 3)
        for _d in range(3):
            wait_out()

    return k(embedding, pospad, comb2d)


def kernel(char_ids, pad_ids, embedding, pos_embedding, padding_embedding):
    pospad = _build_pospad_tc(padding_embedding, pos_embedding)
    pospad = pospad.reshape(NPP, DIM)
    char2d = char_ids.reshape(N // GR, GR).astype(jnp.int32)
    pad2d = pad_ids.reshape(N // GR, GR).astype(jnp.int32)
    comb2d = _pack_ids_tc(char2d, pad2d)
    out = _sc_lookup(embedding, pospad, comb2d)
    return out.reshape(B, S, DIM)


# speedup vs baseline: 2.2432x; 2.0240x over previous
import functools

import jax
import jax.numpy as jnp
from jax import lax
from jax.experimental import pallas as pl
from jax.experimental.pallas import tpu as pltpu
from jax.experimental.pallas import tpu_sc as plsc

B = 4096
S = 200
DIM = 128
N = B * S
NPP = 3 * S

_info = plsc.get_sparse_core_info()
NC, NS, L = _info.num_cores, _info.num_subcores, _info.num_lanes
NW = NC * NS
ROWS_PER_W = N // NW
GR = 128
NG = ROWS_PER_W // GR
IDROWS = ROWS_PER_W // GR


def _build_pospad_tc(padding_embedding, pos_embedding):
    def body(pad_ref, pos_ref, out_ref):
        out_ref[...] = pad_ref[...][:, None, :] + pos_ref[0:S][None, :, :]

    return pl.pallas_call(
        body,
        out_shape=jax.ShapeDtypeStruct((3, S, DIM), jnp.float32),
    )(padding_embedding, pos_embedding)


def _pack_ids_tc(char2d, pad2d):
    def body(char_ref, pad_ref, out_ref):
        out_ref[...] = char_ref[...] * 4 + pad_ref[...]

    return pl.pallas_call(
        body,
        out_shape=jax.ShapeDtypeStruct(char2d.shape, jnp.int32),
    )(char2d, pad2d)


def _sc_lookup(embedding, pospad, comb2d):
    mesh = plsc.VectorSubcoreMesh(core_axis_name="c", subcore_axis_name="s")

    @functools.partial(
        pl.kernel,
        mesh=mesh,
        out_type=jax.ShapeDtypeStruct((N, DIM), jnp.float32),
        scratch_types=[
            pltpu.VMEM((IDROWS, GR), jnp.int32),
            pltpu.VMEM((2, GR), jnp.int32),
            pltpu.VMEM((2, GR), jnp.int32),
            pltpu.VMEM((3, GR, DIM), jnp.float32),
            pltpu.VMEM((2, GR, DIM), jnp.float32),
            pltpu.SemaphoreType.DMA,
            pltpu.SemaphoreType.DMA,
        ],
    )
    def k(emb_hbm, pp_hbm, comb_hbm, out_hbm,
          comb_v, cidx_v, pidx_v, bufa, bufb, sem_g, sem_o):
        wid = lax.axis_index("s") * NC + lax.axis_index("c")
        w_row0 = wid * ROWS_PER_W
        idrow0 = pl.multiple_of(w_row0 // GR, IDROWS)

        pltpu.sync_copy(comb_hbm.at[pl.ds(idrow0, IDROWS)], comb_v)

        lane = lax.iota(jnp.int32, L)

        def build_idx(g, islot):
            for c in range(GR // L):
                cs = pl.ds(c * L, L)
                w = comb_v[g, cs]
                base = (w_row0 + c * L).astype(jnp.int32) + g * GR
                cidx_v[islot, cs] = w >> 2
                pidx_v[islot, cs] = (w & 3) * S + (base + lane) % S

        def gather(g, islot, aslot, bslot):
            pltpu.async_copy(emb_hbm.at[cidx_v.at[islot]],
                             bufa.at[aslot], sem_g)
            pltpu.async_copy(pp_hbm.at[pidx_v.at[islot]],
                             bufb.at[bslot], sem_g)

        def wait_gather(aslot, bslot):
            pltpu.make_async_copy(emb_hbm.at[cidx_v.at[0]],
                                  bufa.at[aslot], sem_g).wait()
            pltpu.make_async_copy(pp_hbm.at[pidx_v.at[0]],
                                  bufb.at[bslot], sem_g).wait()

        def add_group(aslot, bslot):
            def add_row(r, _):
                for c in range(DIM // L):
                    cs = pl.ds(c * L, L)
                    bufa[aslot, r, cs] = (bufa[aslot, r, cs]
                                          + bufb[bslot, r, cs])
                return _
            lax.fori_loop(0, GR, add_row, 0)

        def copy_out(g, aslot):
            pltpu.async_copy(
                bufa.at[aslot], out_hbm.at[pl.ds(w_row0 + g * GR, GR)],
                sem_o)

        def wait_out():
            pltpu.make_async_copy(
                bufa.at[0], out_hbm.at[pl.ds(w_row0, GR)], sem_o).wait()

        build_idx(0, 0)
        gather(0, 0, 0, 0)
        for g in (1, 2):
            build_idx(g, g % 2)
            gather(g, g % 2, g % 3, g % 2)
            wait_gather((g - 1) % 3, (g - 1) % 2)
            add_group((g - 1) % 3, (g - 1) % 2)
            copy_out(g - 1, (g - 1) % 3)

        def steady_group(g, u):
            wait_out()
            build_idx(g, (u + 1) % 2)
            gather(g, (u + 1) % 2, u % 3, (u + 1) % 2)
            wait_gather((u + 2) % 3, u % 2)
            add_group((u + 2) % 3, u % 2)
            copy_out(g - 1, (u + 2) % 3)

        def super_body(kk, _):
            gbase = 3 + 6 * kk
            for u in range(6):
                steady_group(gbase + u, u)
            return _
        lax.fori_loop(0, (NG - 8) // 6, super_body, 0)

        for g in range(3 + 6 * ((NG - 8) // 6), NG):
            steady_group(g, (g - 3) % 6)

        wait_gather((NG - 1) % 3, (NG - 1) % 2)
        add_group((NG - 1) % 3, (NG - 1) % 2)
        copy_out(NG - 1, (NG - 1) % 3)
        for _d in range(3):
            wait_out()

    return k(embedding, pospad, comb2d)


def kernel(char_ids, pad_ids, embedding, pos_embedding, padding_embedding):
    pospad = _build_pospad_tc(padding_embedding, pos_embedding)
    pospad = pospad.reshape(NPP, DIM)
    char2d = char_ids.reshape(N // GR, GR).astype(jnp.int32)
    pad2d = pad_ids.reshape(N // GR, GR).astype(jnp.int32)
    comb2d = _pack_ids_tc(char2d, pad2d)
    out = _sc_lookup(embedding, pospad, comb2d)
    return out.reshape(B, S, DIM)


# speedup vs baseline: 2.2462x; 1.0013x over previous
import functools

import jax
import jax.numpy as jnp
from jax import lax
from jax.experimental import pallas as pl
from jax.experimental.pallas import tpu as pltpu
from jax.experimental.pallas import tpu_sc as plsc

B = 4096
S = 200
DIM = 128
N = B * S
NPP = 3 * S

_info = plsc.get_sparse_core_info()
NC, NS, L = _info.num_cores, _info.num_subcores, _info.num_lanes
NW = NC * NS
ROWS_PER_W = N // NW
GR = 128
NG = ROWS_PER_W // GR
IDROWS = ROWS_PER_W // GR


def _build_pospad_tc(padding_embedding, pos_embedding):
    def body(pad_ref, pos_ref, out_ref):
        out_ref[...] = pad_ref[...][:, None, :] + pos_ref[0:S][None, :, :]

    return pl.pallas_call(
        body,
        out_shape=jax.ShapeDtypeStruct((3, S, DIM), jnp.float32),
    )(padding_embedding, pos_embedding)


def _pack_ids_tc(char2d, pad2d):
    def body(char_ref, pad_ref, out_ref):
        out_ref[...] = char_ref[...] * 4 + pad_ref[...]

    return pl.pallas_call(
        body,
        out_shape=jax.ShapeDtypeStruct(char2d.shape, jnp.int32),
    )(char2d, pad2d)


def _sc_lookup(embedding, pospad, comb2d):
    mesh = plsc.VectorSubcoreMesh(core_axis_name="c", subcore_axis_name="s")

    @functools.partial(
        pl.kernel,
        mesh=mesh,
        out_type=jax.ShapeDtypeStruct((N, DIM), jnp.float32),
        scratch_types=[
            pltpu.VMEM((IDROWS, GR), jnp.int32),
            pltpu.VMEM((2, GR), jnp.int32),
            pltpu.VMEM((2, GR), jnp.int32),
            pltpu.VMEM((3, GR, DIM), jnp.float32),
            pltpu.VMEM((2, GR, DIM), jnp.float32),
            pltpu.SemaphoreType.DMA,
            pltpu.SemaphoreType.DMA,
        ],
    )
    def k(emb_hbm, pp_hbm, comb_hbm, out_hbm,
          comb_v, cidx_v, pidx_v, bufa, bufb, sem_g, sem_o):
        wid = lax.axis_index("s") * NC + lax.axis_index("c")
        w_row0 = wid * ROWS_PER_W
        idrow0 = pl.multiple_of(w_row0 // GR, IDROWS)

        pltpu.sync_copy(comb_hbm.at[pl.ds(idrow0, IDROWS)], comb_v)

        lane = lax.iota(jnp.int32, L)

        def build_idx(g, islot):
            for c in range(GR // L):
                cs = pl.ds(c * L, L)
                w = comb_v[g, cs]
                base = (w_row0 + c * L).astype(jnp.int32) + g * GR
                cidx_v[islot, cs] = w >> 2
                pidx_v[islot, cs] = (w & 3) * S + (base + lane) % S

        def gather(g, islot, aslot, bslot):
            pltpu.async_copy(emb_hbm.at[cidx_v.at[islot]],
                             bufa.at[aslot], sem_g)
            pltpu.async_copy(pp_hbm.at[pidx_v.at[islot]],
                             bufb.at[bslot], sem_g)

        def wait_gather(aslot, bslot):
            pltpu.make_async_copy(emb_hbm.at[cidx_v.at[0]],
                                  bufa.at[aslot], sem_g).wait()
            pltpu.make_async_copy(pp_hbm.at[pidx_v.at[0]],
                                  bufb.at[bslot], sem_g).wait()

        def add_group(aslot, bslot):
            def add_rows(j, _):
                r = pl.multiple_of(j * 2, 2)
                for dr in range(2):
                    for c in range(DIM // L):
                        cs = pl.ds(c * L, L)
                        plsc.addupdate(bufa.at[aslot, r + dr, cs],
                                       bufb[bslot, r + dr, cs])
                return _
            lax.fori_loop(0, GR // 2, add_rows, 0)

        def copy_out(g, aslot):
            pltpu.async_copy(
                bufa.at[aslot], out_hbm.at[pl.ds(w_row0 + g * GR, GR)],
                sem_o)

        def wait_out():
            pltpu.make_async_copy(
                bufa.at[0], out_hbm.at[pl.ds(w_row0, GR)], sem_o).wait()

        build_idx(0, 0)
        gather(0, 0, 0, 0)
        for g in (1, 2):
            build_idx(g, g % 2)
            gather(g, g % 2, g % 3, g % 2)
            wait_gather((g - 1) % 3, (g - 1) % 2)
            add_group((g - 1) % 3, (g - 1) % 2)
            copy_out(g - 1, (g - 1) % 3)

        def steady_group(g, u):
            wait_out()
            build_idx(g, (u + 1) % 2)
            gather(g, (u + 1) % 2, u % 3, (u + 1) % 2)
            wait_gather((u + 2) % 3, u % 2)
            add_group((u + 2) % 3, u % 2)
            copy_out(g - 1, (u + 2) % 3)

        def super_body(kk, _):
            gbase = 3 + 6 * kk
            for u in range(6):
                steady_group(gbase + u, u)
            return _
        lax.fori_loop(0, (NG - 8) // 6, super_body, 0)

        for g in range(3 + 6 * ((NG - 8) // 6), NG):
            steady_group(g, (g - 3) % 6)

        wait_gather((NG - 1) % 3, (NG - 1) % 2)
        add_group((NG - 1) % 3, (NG - 1) % 2)
        copy_out(NG - 1, (NG - 1) % 3)
        for _d in range(3):
            wait_out()

    return k(embedding, pospad, comb2d)


def kernel(char_ids, pad_ids, embedding, pos_embedding, padding_embedding):
    pospad = _build_pospad_tc(padding_embedding, pos_embedding)
    pospad = pospad.reshape(NPP, DIM)
    char2d = char_ids.reshape(N // GR, GR).astype(jnp.int32)
    pad2d = pad_ids.reshape(N // GR, GR).astype(jnp.int32)
    comb2d = _pack_ids_tc(char2d, pad2d)
    out = _sc_lookup(embedding, pospad, comb2d)
    return out.reshape(B, S, DIM)
